# baseline (speedup 1.0000x reference)
import functools

import jax
import jax.numpy as jnp
from jax.experimental import pallas as pl
from jax.experimental.pallas import tpu as pltpu

D_OUT = 2048
D_IN = 2048
BAND = 256
SUP = 1024
NSUP = D_IN // SUP
NPLANE = 4
SRC = D_IN // 2


def _band_kernel(xb, ib, c0p, c1p, p0p, p1p, out_ref, h_ref):
    band = pl.program_id(0)
    r0 = band * BAND

    @pl.when(band == 0)
    def _build_h():
        slh = jax.lax.broadcasted_iota(jnp.int32, (SUP // 2, 1), 0) >> 1
        for s in range(NSUP):
            accs = [jnp.zeros((SUP // 2, SUP), dtype=jnp.float32)
                    for _ in range(NPLANE)]
            for j in range(NPLANE):
                pm = p1p[0, j:j + 1, s * SUP:(s + 1) * SUP]
                cm = c1p[0, j:j + 1, s * SUP:(s + 1) * SUP]
                hit = (pm >> 2) == s * (SUP // 4) + slh
                for p in range(NPLANE):
                    cmp_ = jnp.where((pm & 3) == p, cm, 0.0)
                    accs[p] = accs[p] + jnp.where(hit, cmp_, 0.0)
            for p in range(NPLANE):
                h_ref[p, s] = accs[p]

    rowid = jax.lax.broadcasted_iota(jnp.int32, (BAND, SRC), 0)
    sid = jax.lax.broadcasted_iota(jnp.int32, (BAND, SRC), 1)
    base = (r0 + rowid) * D_IN + 4 * (sid >> 1)
    off = ib[...] - base
    xv = xb[...]

    parts = []
    for s in range(NSUP):
        acc = None
        for p in range(NPLANE):
            u = jnp.where(off[:, s * (SUP // 2):(s + 1) * (SUP // 2)] == p,
                          xv[:, s * (SUP // 2):(s + 1) * (SUP // 2)], 0.0)
            d = jnp.dot(u, h_ref[p, s], preferred_element_type=jnp.float32)
            acc = d if acc is None else acc + d
        parts.append(acc)
    v = jnp.concatenate(parts, axis=1)

    jrel = jax.lax.broadcasted_iota(jnp.int32, (64, 1), 0)
    jl = jax.lax.broadcasted_iota(jnp.int32, (1, BAND), 1)
    tgt0 = r0 + (jl & ~63) + jrel
    ntb = jnp.zeros((64, BAND), dtype=jnp.float32)
    for j in range(NPLANE):
        po = p0p[0, j:j + 1, :]
        co = c0p[0, j:j + 1, :]
        ntb = ntb + jnp.where(po == tgt0, co, 0.0)
    for c in range(BAND // 64):
        out_ref[c * 64:(c + 1) * 64, :] = jax.lax.dot_general(
            ntb[:, c * 64:(c + 1) * 64], v[c * 64:(c + 1) * 64, :],
            (((0,), (0,)), ((), ())), preferred_element_type=jnp.float32)


@functools.partial(jax.jit, static_argnames=("interpret",))
def kernel(X, c_0, c_1, mask_idx, perm0, perm1, interpret=False):
    xb = X.reshape(D_OUT, SRC)
    ib = mask_idx.reshape(D_OUT, SRC)
    c0p = c_0.reshape(1, 4, D_OUT)
    c1p = c_1.reshape(1, 4, D_IN)
    p0p = perm0.reshape(1, 4, D_OUT)
    p1p = perm1.reshape(1, 4, D_IN)

    return pl.pallas_call(
        _band_kernel,
        grid=(D_OUT // BAND,),
        in_specs=[
            pl.BlockSpec((BAND, SRC), lambda i: (i, 0)),
            pl.BlockSpec((BAND, SRC), lambda i: (i, 0)),
            pl.BlockSpec((1, 4, BAND), lambda i: (0, 0, i)),
            pl.BlockSpec((1, 4, D_IN), lambda i: (0, 0, 0)),
            pl.BlockSpec((1, 4, BAND), lambda i: (0, 0, i)),
            pl.BlockSpec((1, 4, D_IN), lambda i: (0, 0, 0)),
        ],
        out_specs=pl.BlockSpec((BAND, D_IN), lambda i: (i, 0)),
        out_shape=jax.ShapeDtypeStruct((D_OUT, D_IN), jnp.float32),
        scratch_shapes=[
            pltpu.VMEM((NPLANE, NSUP, SUP // 2, SUP), jnp.float32)],
        interpret=interpret,
    )(xb, ib, c0p, c1p, p0p, p1p)


# speedup vs baseline: 1.2790x; 1.2790x over previous
import functools

import jax
import jax.numpy as jnp
from jax.experimental import pallas as pl
from jax.experimental.pallas import tpu as pltpu

D_OUT = 2048
D_IN = 2048
BAND = 256
SUP = 256
NSUP = D_IN // SUP
NPLANE = 4
SRC = D_IN // 2


def _band_kernel(xb, ib, c0p, c1p, p0p, p1p, out_ref, h_ref):
    band = pl.program_id(0)
    r0 = band * BAND

    @pl.when(band == 0)
    def _build_h():
        slh = jax.lax.broadcasted_iota(jnp.int32, (SUP // 2, 1), 0) >> 1
        for s in range(NSUP):
            accs = [jnp.zeros((SUP // 2, SUP), dtype=jnp.float32)
                    for _ in range(NPLANE)]
            for j in range(NPLANE):
                pm = p1p[0, j:j + 1, s * SUP:(s + 1) * SUP]
                cm = c1p[0, j:j + 1, s * SUP:(s + 1) * SUP]
                hit = (pm >> 2) == s * (SUP // 4) + slh
                for p in range(NPLANE):
                    cmp_ = jnp.where((pm & 3) == p, cm, 0.0)
                    accs[p] = accs[p] + jnp.where(hit, cmp_, 0.0)
            for p in range(NPLANE):
                h_ref[p, s] = accs[p]

    rowid = jax.lax.broadcasted_iota(jnp.int32, (BAND, SRC), 0)
    sid = jax.lax.broadcasted_iota(jnp.int32, (BAND, SRC), 1)
    base = (r0 + rowid) * D_IN + 4 * (sid >> 1)
    off = ib[...] - base
    xv = xb[...]

    parts = []
    for s in range(NSUP):
        acc = None
        for p in range(NPLANE):
            u = jnp.where(off[:, s * (SUP // 2):(s + 1) * (SUP // 2)] == p,
                          xv[:, s * (SUP // 2):(s + 1) * (SUP // 2)], 0.0)
            d = jnp.dot(u, h_ref[p, s], preferred_element_type=jnp.float32)
            acc = d if acc is None else acc + d
        parts.append(acc)
    v = jnp.concatenate(parts, axis=1)

    jrel = jax.lax.broadcasted_iota(jnp.int32, (64, 1), 0)
    jl = jax.lax.broadcasted_iota(jnp.int32, (1, BAND), 1)
    tgt0 = r0 + (jl & ~63) + jrel
    ntb = jnp.zeros((64, BAND), dtype=jnp.float32)
    for j in range(NPLANE):
        po = p0p[0, j:j + 1, :]
        co = c0p[0, j:j + 1, :]
        ntb = ntb + jnp.where(po == tgt0, co, 0.0)
    for c in range(BAND // 64):
        out_ref[c * 64:(c + 1) * 64, :] = jax.lax.dot_general(
            ntb[:, c * 64:(c + 1) * 64], v[c * 64:(c + 1) * 64, :],
            (((0,), (0,)), ((), ())), preferred_element_type=jnp.float32)


@functools.partial(jax.jit, static_argnames=("interpret",))
def kernel(X, c_0, c_1, mask_idx, perm0, perm1, interpret=False):
    xb = X.reshape(D_OUT, SRC)
    ib = mask_idx.reshape(D_OUT, SRC)
    c0p = c_0.reshape(1, 4, D_OUT)
    c1p = c_1.reshape(1, 4, D_IN)
    p0p = perm0.reshape(1, 4, D_OUT)
    p1p = perm1.reshape(1, 4, D_IN)

    return pl.pallas_call(
        _band_kernel,
        grid=(D_OUT // BAND,),
        in_specs=[
            pl.BlockSpec((BAND, SRC), lambda i: (i, 0)),
            pl.BlockSpec((BAND, SRC), lambda i: (i, 0)),
            pl.BlockSpec((1, 4, BAND), lambda i: (0, 0, i)),
            pl.BlockSpec((1, 4, D_IN), lambda i: (0, 0, 0)),
            pl.BlockSpec((1, 4, BAND), lambda i: (0, 0, i)),
            pl.BlockSpec((1, 4, D_IN), lambda i: (0, 0, 0)),
        ],
        out_specs=pl.BlockSpec((BAND, D_IN), lambda i: (i, 0)),
        out_shape=jax.ShapeDtypeStruct((D_OUT, D_IN), jnp.float32),
        scratch_shapes=[
            pltpu.VMEM((NPLANE, NSUP, SUP // 2, SUP), jnp.float32)],
        interpret=interpret,
    )(xb, ib, c0p, c1p, p0p, p1p)
